# baseline (speedup 1.0000x reference)
import functools

import jax
import jax.numpy as jnp
from jax import lax
from jax.experimental import pallas as pl
from jax.experimental.pallas import tpu as pltpu
from jax.experimental.pallas import tpu_sc as plsc

K = 17
H = 128
W = 128
LANES = 16
CPR = W // LANES
INPUT_SIZE = 512
SCALE = float(INPUT_SIZE) / W
CONF_THRESHOLD = 0.8

_mesh = plsc.VectorSubcoreMesh(core_axis_name="c", subcore_axis_name="s")


@functools.partial(
    pl.kernel,
    out_type=jax.ShapeDtypeStruct((K, LANES), jnp.float32),
    mesh=_mesh,
    scratch_types=[
        pltpu.VMEM((4, H // 4, W), jnp.float32),
        pltpu.VMEM((LANES,), jnp.float32),
        pltpu.SemaphoreType.DMA,
        pltpu.SemaphoreType.DMA,
        pltpu.SemaphoreType.DMA,
        pltpu.SemaphoreType.DMA,
    ],
    compiler_params=pltpu.CompilerParams(
        needs_layout_passes=False, skip_device_barrier=True
    ),
)
def _decode_sc(x_hbm, out_hbm, xv, res_v, s0, s1, s2, s3):
    wid = lax.axis_index("c") * 16 + lax.axis_index("s")
    QR = H // 4

    def _process(kp):
        sems = (s0, s1, s2, s3)
        copies = [
            pltpu.async_copy(
                x_hbm.at[kp, pl.ds(q * QR, QR)], xv.at[q], sems[q]
            )
            for q in range(4)
        ]
        lane = lax.iota(jnp.int32, 16)

        vmax0 = jnp.full((LANES,), -jnp.inf, jnp.float32)
        vidx0 = jnp.zeros((LANES,), jnp.int32)

        accs = tuple((vmax0, vidx0) for _ in range(4))
        for q in range(4):
            copies[q].wait()

            def row_body_q(r, carry, q=q):
                accs = list(carry)
                rowbase = lane + (r + q * QR) * W
                for u in range(CPR):
                    a = u % 4
                    vmax, vidx = accs[a]
                    v = xv[q, r, pl.ds(u * LANES, LANES)]
                    m = v > vmax
                    accs[a] = (
                        jnp.where(m, v, vmax),
                        jnp.where(m, rowbase + u * LANES, vidx),
                    )
                return tuple(accs)

            accs = lax.fori_loop(0, QR, row_body_q, accs)

        (vmax, vidx) = accs[0]
        for a in range(1, 4):
            pm, pi = accs[a]
            take = (pm > vmax) | ((pm == vmax) & (pi < vidx))
            vmax = jnp.where(take, pm, vmax)
            vidx = jnp.where(take, pi, vidx)

        gmax = jnp.max(vmax)
        cand = jnp.where(vmax == gmax, vidx, jnp.int32(1 << 30))
        gidx = jnp.min(cand)

        gmax_v = jnp.full((LANES,), gmax, jnp.float32)
        gidx_v = jnp.full((LANES,), gidx, jnp.int32)
        conf_v = 1.0 / (1.0 + jnp.exp(-gmax_v))
        xx_v = (gidx_v % W).astype(jnp.float32) * SCALE
        yy_v = (gidx_v // W).astype(jnp.float32) * SCALE
        res = jnp.where(
            lane == 0,
            xx_v,
            jnp.where(lane == 1, yy_v, jnp.where(lane == 2, conf_v, -1.0)),
        )
        nodet = jnp.where(lane == 2, -1.0, -1.0 * SCALE)
        res = jnp.where(conf_v > CONF_THRESHOLD, res, nodet)
        res_v[...] = res
        pltpu.sync_copy(res_v, out_hbm.at[kp])

    @pl.when(wid < K)
    def _():
        _process(wid)


def kernel(x):
    out = _decode_sc(x[0])
    return out[:, :3]


# speedup vs baseline: 1.0008x; 1.0008x over previous
import functools

import jax
import jax.numpy as jnp
from jax import lax
from jax.experimental import pallas as pl
from jax.experimental.pallas import tpu as pltpu
from jax.experimental.pallas import tpu_sc as plsc

K = 17
H = 128
W = 128
LANES = 16
CPR = W // LANES
INPUT_SIZE = 512
SCALE = float(INPUT_SIZE) / W
CONF_THRESHOLD = 0.8

_mesh = plsc.VectorSubcoreMesh(core_axis_name="c", subcore_axis_name="s")


@functools.partial(
    pl.kernel,
    out_type=jax.ShapeDtypeStruct((K, LANES), jnp.float32),
    mesh=_mesh,
    scratch_types=[
        pltpu.VMEM((4, H // 4, W), jnp.float32),
        pltpu.VMEM((LANES,), jnp.float32),
        pltpu.SemaphoreType.DMA,
        pltpu.SemaphoreType.DMA,
        pltpu.SemaphoreType.DMA,
        pltpu.SemaphoreType.DMA,
    ],
    compiler_params=pltpu.CompilerParams(needs_layout_passes=False),
)
def _decode_sc(x_hbm, out_hbm, xv, res_v, s0, s1, s2, s3):
    wid = lax.axis_index("c") * 16 + lax.axis_index("s")
    QR = H // 4

    def _process(kp):
        sems = (s0, s1, s2, s3)
        copies = [
            pltpu.async_copy(
                x_hbm.at[kp, pl.ds(q * QR, QR)], xv.at[q], sems[q]
            )
            for q in range(4)
        ]
        lane = lax.iota(jnp.int32, 16)

        vmax0 = jnp.full((LANES,), -jnp.inf, jnp.float32)
        vidx0 = jnp.zeros((LANES,), jnp.int32)

        accs = tuple((vmax0, vidx0) for _ in range(4))
        for q in range(4):
            copies[q].wait()

            def row_body_q(r, carry, q=q):
                accs = list(carry)
                rowbase = lane + (r + q * QR) * W
                for u in range(CPR):
                    a = u % 4
                    vmax, vidx = accs[a]
                    v = xv[q, r, pl.ds(u * LANES, LANES)]
                    m = v > vmax
                    accs[a] = (
                        jnp.where(m, v, vmax),
                        jnp.where(m, rowbase + u * LANES, vidx),
                    )
                return tuple(accs)

            accs = lax.fori_loop(0, QR, row_body_q, accs)

        (vmax, vidx) = accs[0]
        for a in range(1, 4):
            pm, pi = accs[a]
            take = (pm > vmax) | ((pm == vmax) & (pi < vidx))
            vmax = jnp.where(take, pm, vmax)
            vidx = jnp.where(take, pi, vidx)

        gmax = jnp.max(vmax)
        cand = jnp.where(vmax == gmax, vidx, jnp.int32(1 << 30))
        gidx = jnp.min(cand)

        gmax_v = jnp.full((LANES,), gmax, jnp.float32)
        gidx_v = jnp.full((LANES,), gidx, jnp.int32)
        conf_v = 1.0 / (1.0 + jnp.exp(-gmax_v))
        xx_v = (gidx_v % W).astype(jnp.float32) * SCALE
        yy_v = (gidx_v // W).astype(jnp.float32) * SCALE
        res = jnp.where(
            lane == 0,
            xx_v,
            jnp.where(lane == 1, yy_v, jnp.where(lane == 2, conf_v, -1.0)),
        )
        nodet = jnp.where(lane == 2, -1.0, -1.0 * SCALE)
        res = jnp.where(conf_v > CONF_THRESHOLD, res, nodet)
        res_v[...] = res
        pltpu.sync_copy(res_v, out_hbm.at[kp])

    @pl.when(wid < K)
    def _():
        _process(wid)


def kernel(x):
    out = _decode_sc(x[0])
    return out[:, :3]
